# baseline (speedup 1.0000x reference)
import functools

import jax
import jax.numpy as jnp
from jax import lax
from jax.experimental import pallas as pl
import jax.experimental.pallas.tpu as pltpu
from jax.experimental.pallas import tpu_sc as plsc

E = 8
TOPK = 2
BLK = 256
MAXB = 24
NW = 32


def _router_kernel(x_ref, wr_ref, br_ref,
                   pos0_ref, pos1_ref, w0_ref, w1_ref, be_ref, nb_ref,
                   xbf_ref):
    T = x_ref.shape[0]
    xf = x_ref[...]
    xbf_ref[...] = xf.astype(jnp.bfloat16)
    logits = jnp.dot(xf, wr_ref[...],
                     preferred_element_type=jnp.float32) + br_ref[...]
    m = jnp.max(logits, axis=-1, keepdims=True)
    ex = jnp.exp(logits - m)
    probs = ex / jnp.sum(ex, axis=-1, keepdims=True)
    col = lax.broadcasted_iota(jnp.int32, probs.shape, 1)
    big = jnp.int32(E + 1)
    m1 = jnp.max(probs, axis=-1, keepdims=True)
    a1 = jnp.min(jnp.where(probs == m1, col, big), axis=-1, keepdims=True)
    p2 = jnp.where(col == a1, -jnp.inf, probs)
    m2 = jnp.max(p2, axis=-1, keepdims=True)
    a2 = jnp.min(jnp.where(p2 == m2, col, big), axis=-1, keepdims=True)
    denom = m1 + m2
    w0_ref[...] = m1 / denom
    w1_ref[...] = m2 / denom

    Mm = ((col == a1) | (col == a2)).astype(jnp.bfloat16)
    r0 = lax.broadcasted_iota(jnp.int32, (T, T), 0)
    r1 = lax.broadcasted_iota(jnp.int32, (T, T), 1)
    L = (r1 < r0).astype(jnp.bfloat16)
    cnt_excl = jnp.dot(L, Mm, preferred_element_type=jnp.float32)

    counts = jnp.sum(Mm.astype(jnp.float32), axis=0, keepdims=True)
    nb = jnp.floor((counts + (BLK - 1)) / BLK)
    ecol0 = lax.broadcasted_iota(jnp.int32, (E, E), 0)
    ecol1 = lax.broadcasted_iota(jnp.int32, (E, E), 1)
    SU = (ecol0 < ecol1).astype(jnp.float32)
    offs_row = BLK * jnp.dot(nb, SU,
                             preferred_element_type=jnp.float32)

    posf0 = jnp.sum(jnp.where(col == a1, cnt_excl + offs_row, 0.0),
                    axis=-1, keepdims=True)
    posf1 = jnp.sum(jnp.where(col == a2, cnt_excl + offs_row, 0.0),
                    axis=-1, keepdims=True)
    pos0_ref[...] = posf0.astype(jnp.int32)
    pos1_ref[...] = posf1.astype(jnp.int32)

    IdE = (ecol0 == ecol1).astype(jnp.float32)
    SL = (ecol1 < ecol0).astype(jnp.float32)
    nbc = lax.dot_general(IdE, nb, (((1,), (1,)), ((), ())),
                          preferred_element_type=jnp.float32)
    startblk = jnp.dot(SL, nbc, preferred_element_type=jnp.float32)
    total = jnp.sum(nb, axis=-1, keepdims=True)
    bio = lax.broadcasted_iota(jnp.int32, (1, MAXB), 1).astype(jnp.float32)
    bclamp = jnp.minimum(bio, total - 1.0)
    owners = jnp.sum((startblk <= bclamp).astype(jnp.float32),
                     axis=0, keepdims=True)
    be_ref[...] = (owners - 1.0).astype(jnp.int32)
    nb_ref[...] = total.astype(jnp.int32)


def _router(x2d, Wr, br2):
    T, D = x2d.shape
    outs = pl.pallas_call(
        _router_kernel,
        grid=(1,),
        in_specs=[
            pl.BlockSpec((T, D), lambda i: (0, 0)),
            pl.BlockSpec((D, E), lambda i: (0, 0)),
            pl.BlockSpec((1, E), lambda i: (0, 0)),
        ],
        out_specs=[
            pl.BlockSpec((T, 1), lambda i: (0, 0)),
            pl.BlockSpec((T, 1), lambda i: (0, 0)),
            pl.BlockSpec((T, 1), lambda i: (0, 0)),
            pl.BlockSpec((T, 1), lambda i: (0, 0)),
            pl.BlockSpec((1, MAXB), lambda i: (0, 0)),
            pl.BlockSpec((1, 1), lambda i: (0, 0)),
            pl.BlockSpec((T, D), lambda i: (0, 0)),
        ],
        out_shape=[
            jax.ShapeDtypeStruct((T, 1), jnp.int32),
            jax.ShapeDtypeStruct((T, 1), jnp.int32),
            jax.ShapeDtypeStruct((T, 1), jnp.float32),
            jax.ShapeDtypeStruct((T, 1), jnp.float32),
            jax.ShapeDtypeStruct((1, MAXB), jnp.int32),
            jax.ShapeDtypeStruct((1, 1), jnp.int32),
            jax.ShapeDtypeStruct((T, D), jnp.bfloat16),
        ],
    )(x2d, Wr, br2)
    return outs


def _sc_scatter(x2d, i0, i1, pbuf):
    T, D = x2d.shape
    bpw = T // NW
    mesh = plsc.VectorSubcoreMesh(core_axis_name="c", subcore_axis_name="s")

    @functools.partial(
        pl.kernel, mesh=mesh,
        out_type=jax.ShapeDtypeStruct((pbuf, D), jnp.int32),
        scratch_types=[
            pltpu.VMEM((bpw,), jnp.int32),
            pltpu.VMEM((bpw,), jnp.int32),
            pltpu.VMEM((bpw, D), jnp.int32),
            pltpu.SemaphoreType.DMA,
        ],
    )
    def scat(x_hbm, i0_hbm, i1_hbm, xg_hbm, i0_v, i1_v, rows_v, sem):
        wid = lax.axis_index("s") * 2 + lax.axis_index("c")
        base = wid * bpw
        pltpu.sync_copy(i0_hbm.at[wid], i0_v)
        pltpu.sync_copy(i1_hbm.at[wid], i1_v)
        pltpu.sync_copy(x_hbm.at[pl.ds(base, bpw)], rows_v)
        pltpu.async_copy(rows_v, xg_hbm.at[i0_v], sem).wait()
        pltpu.async_copy(rows_v, xg_hbm.at[i1_v], sem).wait()

    return scat(x2d, i0, i1)


def _sc_gather(yg, i0, i1):
    pbuf, D = yg.shape
    T = i0.shape[0] * i0.shape[1]
    bpw = T // NW
    mesh = plsc.VectorSubcoreMesh(core_axis_name="c", subcore_axis_name="s")

    @functools.partial(
        pl.kernel, mesh=mesh,
        out_type=(jax.ShapeDtypeStruct((T, D), jnp.float32),
                  jax.ShapeDtypeStruct((T, D), jnp.float32)),
        scratch_types=[
            pltpu.VMEM((bpw,), jnp.int32),
            pltpu.VMEM((bpw,), jnp.int32),
            pltpu.VMEM((bpw, D), jnp.float32),
            pltpu.SemaphoreType.DMA,
        ],
    )
    def gath(yg_hbm, i0_hbm, i1_hbm, o0_hbm, o1_hbm, i0_v, i1_v, rows_v, sem):
        wid = lax.axis_index("s") * 2 + lax.axis_index("c")
        base = wid * bpw
        pltpu.sync_copy(i0_hbm.at[wid], i0_v)
        pltpu.sync_copy(i1_hbm.at[wid], i1_v)
        pltpu.async_copy(yg_hbm.at[i0_v], rows_v, sem).wait()
        pltpu.sync_copy(rows_v, o0_hbm.at[pl.ds(base, bpw)])
        pltpu.async_copy(yg_hbm.at[i1_v], rows_v, sem).wait()
        pltpu.sync_copy(rows_v, o1_hbm.at[pl.ds(base, bpw)])

    return gath(yg, i0, i1)


def _expert_changed(be_ref, nb_ref):
    b = pl.program_id(0)
    e_prev = jnp.where(b == 0, -1, be_ref[jnp.maximum(b - 1, 0)])
    return jnp.logical_and(b < nb_ref[0], be_ref[b] != e_prev)


def _ffn1_kernel(be_ref, nb_ref, xg_ref, w1_ref, b1_ref, h_ref, w1bf_ref):
    b = pl.program_id(0)

    @pl.when(_expert_changed(be_ref, nb_ref))
    def _cast():
        w1bf_ref[...] = w1_ref[0].astype(jnp.bfloat16)

    @pl.when(b < nb_ref[0])
    def _():
        h = jnp.dot(xg_ref[...], w1bf_ref[...],
                    preferred_element_type=jnp.float32) + b1_ref[0]
        h_ref[...] = jnp.maximum(h, 0.0).astype(jnp.bfloat16)


def _ffn2_kernel(be_ref, nb_ref, h_ref, w2_ref, b2_ref, yg_ref, w2bf_ref):
    b = pl.program_id(0)

    @pl.when(_expert_changed(be_ref, nb_ref))
    def _cast():
        w2bf_ref[...] = w2_ref[0].astype(jnp.bfloat16)

    @pl.when(b < nb_ref[0])
    def _():
        yg_ref[...] = jnp.dot(h_ref[...], w2bf_ref[...],
                              preferred_element_type=jnp.float32) + b2_ref[0]


def _ffn(xg, W1, b1r, W2, b2r, be, nb):
    pbuf, D = xg.shape
    F = W1.shape[2]
    H = pl.pallas_call(
        _ffn1_kernel,
        grid_spec=pltpu.PrefetchScalarGridSpec(
            num_scalar_prefetch=2,
            grid=(MAXB,),
            in_specs=[
                pl.BlockSpec((BLK, D), lambda b, be, nb: (b, 0)),
                pl.BlockSpec((1, D, F), lambda b, be, nb: (be[b], 0, 0)),
                pl.BlockSpec((1, 1, F), lambda b, be, nb: (be[b], 0, 0)),
            ],
            out_specs=pl.BlockSpec((BLK, F), lambda b, be, nb: (b, 0)),
            scratch_shapes=[pltpu.VMEM((D, F), jnp.bfloat16)],
        ),
        out_shape=jax.ShapeDtypeStruct((pbuf, F), jnp.bfloat16),
    )(be, nb, xg, W1, b1r)
    yg = pl.pallas_call(
        _ffn2_kernel,
        grid_spec=pltpu.PrefetchScalarGridSpec(
            num_scalar_prefetch=2,
            grid=(MAXB,),
            in_specs=[
                pl.BlockSpec((BLK, F), lambda b, be, nb: (b, 0)),
                pl.BlockSpec((1, F, D), lambda b, be, nb: (be[b], 0, 0)),
                pl.BlockSpec((1, 1, D), lambda b, be, nb: (be[b], 0, 0)),
            ],
            out_specs=pl.BlockSpec((BLK, D), lambda b, be, nb: (b, 0)),
            scratch_shapes=[pltpu.VMEM((F, D), jnp.bfloat16)],
        ),
        out_shape=jax.ShapeDtypeStruct((pbuf, D), jnp.float32),
    )(be, nb, H, W2, b2r)
    return yg


def _combine_kernel(y0_ref, y1_ref, w0_ref, w1_ref, out_ref):
    out_ref[...] = y0_ref[...] * w0_ref[...] + y1_ref[...] * w1_ref[...]


def _combine(yg0, yg1, w0, w1):
    T, D = yg0.shape
    BT = 512
    return pl.pallas_call(
        _combine_kernel,
        grid=(T // BT,),
        in_specs=[
            pl.BlockSpec((BT, D), lambda i: (i, 0)),
            pl.BlockSpec((BT, D), lambda i: (i, 0)),
            pl.BlockSpec((BT, 1), lambda i: (i, 0)),
            pl.BlockSpec((BT, 1), lambda i: (i, 0)),
        ],
        out_specs=pl.BlockSpec((BT, D), lambda i: (i, 0)),
        out_shape=jax.ShapeDtypeStruct((T, D), jnp.float32),
    )(yg0, yg1, w0, w1)


@jax.jit
def _moe(x2d, Wr, br2, W1, b1r, W2, b2r):
    T, D = x2d.shape
    pbuf = MAXB * BLK
    pos0, pos1, w0, w1, be, nb, xbf = _router(x2d, Wr, br2)
    i0 = pos0.reshape(NW, T // NW)
    i1 = pos1.reshape(NW, T // NW)
    xi = lax.bitcast_convert_type(xbf.reshape(T, D // 2, 2), jnp.int32)
    xgi = _sc_scatter(xi, i0, i1, pbuf)
    xg = lax.bitcast_convert_type(xgi, jnp.bfloat16).reshape(pbuf, D)
    yg = _ffn(xg, W1, b1r, W2, b2r, be.reshape(MAXB), nb.reshape(1))
    yg0, yg1 = _sc_gather(yg, i0, i1)
    return _combine(yg0, yg1, w0, w1)


def kernel(x, Wr, br, W1, b1, W2, b2):
    B, S, D = x.shape
    x2d = x.reshape(B * S, D)
    out = _moe(x2d, Wr, br.reshape(1, E),
               W1, b1.reshape(E, 1, -1), W2, b2.reshape(E, 1, -1))
    return out.reshape(B, S, D)


# speedup vs baseline: 1.6423x; 1.6423x over previous
import functools

import jax
import jax.numpy as jnp
from jax import lax
from jax.experimental import pallas as pl
import jax.experimental.pallas.tpu as pltpu
from jax.experimental.pallas import tpu_sc as plsc

E = 8
TOPK = 2
BLK = 256
MAXB = 24
NW = 32


def _router_kernel(x_ref, wr_ref, br_ref,
                   pos0_ref, pos1_ref, w0_ref, w1_ref, be_ref, nb_ref,
                   xbf_ref):
    T = x_ref.shape[0]
    xf = x_ref[...]
    xbf_ref[...] = xf.astype(jnp.bfloat16)
    logits = jnp.dot(xf, wr_ref[...],
                     preferred_element_type=jnp.float32) + br_ref[...]
    m = jnp.max(logits, axis=-1, keepdims=True)
    ex = jnp.exp(logits - m)
    probs = ex / jnp.sum(ex, axis=-1, keepdims=True)
    col = lax.broadcasted_iota(jnp.int32, probs.shape, 1)
    big = jnp.int32(E + 1)
    m1 = jnp.max(probs, axis=-1, keepdims=True)
    a1 = jnp.min(jnp.where(probs == m1, col, big), axis=-1, keepdims=True)
    p2 = jnp.where(col == a1, -jnp.inf, probs)
    m2 = jnp.max(p2, axis=-1, keepdims=True)
    a2 = jnp.min(jnp.where(p2 == m2, col, big), axis=-1, keepdims=True)
    denom = m1 + m2
    w0_ref[...] = m1 / denom
    w1_ref[...] = m2 / denom

    Mm = ((col == a1) | (col == a2)).astype(jnp.bfloat16)
    r0 = lax.broadcasted_iota(jnp.int32, (T, T), 0)
    r1 = lax.broadcasted_iota(jnp.int32, (T, T), 1)
    L = (r1 < r0).astype(jnp.bfloat16)
    cnt_excl = jnp.dot(L, Mm, preferred_element_type=jnp.float32)

    counts = jnp.sum(Mm.astype(jnp.float32), axis=0, keepdims=True)
    nb = jnp.floor((counts + (BLK - 1)) / BLK)
    ecol0 = lax.broadcasted_iota(jnp.int32, (E, E), 0)
    ecol1 = lax.broadcasted_iota(jnp.int32, (E, E), 1)
    SU = (ecol0 < ecol1).astype(jnp.float32)
    offs_row = BLK * jnp.dot(nb, SU,
                             preferred_element_type=jnp.float32)

    posf0 = jnp.sum(jnp.where(col == a1, cnt_excl + offs_row, 0.0),
                    axis=-1, keepdims=True)
    posf1 = jnp.sum(jnp.where(col == a2, cnt_excl + offs_row, 0.0),
                    axis=-1, keepdims=True)
    pos0_ref[...] = posf0.astype(jnp.int32)
    pos1_ref[...] = posf1.astype(jnp.int32)

    IdE = (ecol0 == ecol1).astype(jnp.float32)
    SL = (ecol1 < ecol0).astype(jnp.float32)
    nbc = lax.dot_general(IdE, nb, (((1,), (1,)), ((), ())),
                          preferred_element_type=jnp.float32)
    startblk = jnp.dot(SL, nbc, preferred_element_type=jnp.float32)
    total = jnp.sum(nb, axis=-1, keepdims=True)
    bio = lax.broadcasted_iota(jnp.int32, (1, MAXB), 1).astype(jnp.float32)
    bclamp = jnp.minimum(bio, total - 1.0)
    owners = jnp.sum((startblk <= bclamp).astype(jnp.float32),
                     axis=0, keepdims=True)
    be_ref[...] = (owners - 1.0).astype(jnp.int32)
    nb_ref[...] = total.astype(jnp.int32)


def _router(x2d, Wr, br2):
    T, D = x2d.shape
    outs = pl.pallas_call(
        _router_kernel,
        grid=(1,),
        in_specs=[
            pl.BlockSpec((T, D), lambda i: (0, 0)),
            pl.BlockSpec((D, E), lambda i: (0, 0)),
            pl.BlockSpec((1, E), lambda i: (0, 0)),
        ],
        out_specs=[
            pl.BlockSpec((T, 1), lambda i: (0, 0)),
            pl.BlockSpec((T, 1), lambda i: (0, 0)),
            pl.BlockSpec((T, 1), lambda i: (0, 0)),
            pl.BlockSpec((T, 1), lambda i: (0, 0)),
            pl.BlockSpec((1, MAXB), lambda i: (0, 0)),
            pl.BlockSpec((1, 1), lambda i: (0, 0)),
            pl.BlockSpec((T, D), lambda i: (0, 0)),
        ],
        out_shape=[
            jax.ShapeDtypeStruct((T, 1), jnp.int32),
            jax.ShapeDtypeStruct((T, 1), jnp.int32),
            jax.ShapeDtypeStruct((T, 1), jnp.float32),
            jax.ShapeDtypeStruct((T, 1), jnp.float32),
            jax.ShapeDtypeStruct((1, MAXB), jnp.int32),
            jax.ShapeDtypeStruct((1, 1), jnp.int32),
            jax.ShapeDtypeStruct((T, D), jnp.bfloat16),
        ],
    )(x2d, Wr, br2)
    return outs


def _sc_scatter(x2d, i0, i1, pbuf):
    T, D = x2d.shape
    bpw = T // NW
    mesh = plsc.VectorSubcoreMesh(core_axis_name="c", subcore_axis_name="s")

    @functools.partial(
        pl.kernel, mesh=mesh,
        out_type=jax.ShapeDtypeStruct((pbuf, D), jnp.float32),
        scratch_types=[
            pltpu.VMEM((bpw,), jnp.int32),
            pltpu.VMEM((bpw,), jnp.int32),
            pltpu.VMEM((bpw, D), jnp.float32),
            pltpu.SemaphoreType.DMA,
        ],
    )
    def scat(x_hbm, i0_hbm, i1_hbm, xg_hbm, i0_v, i1_v, rows_v, sem):
        wid = lax.axis_index("s") * 2 + lax.axis_index("c")
        base = wid * bpw
        pltpu.sync_copy(i0_hbm.at[wid], i0_v)
        pltpu.sync_copy(i1_hbm.at[wid], i1_v)
        pltpu.sync_copy(x_hbm.at[pl.ds(base, bpw)], rows_v)
        pltpu.async_copy(rows_v, xg_hbm.at[i0_v], sem).wait()
        pltpu.async_copy(rows_v, xg_hbm.at[i1_v], sem).wait()

    return scat(x2d, i0, i1)


def _sc_gather(yg, i0, i1):
    pbuf, D = yg.shape
    T = i0.shape[0] * i0.shape[1]
    bpw = T // NW
    mesh = plsc.VectorSubcoreMesh(core_axis_name="c", subcore_axis_name="s")

    @functools.partial(
        pl.kernel, mesh=mesh,
        out_type=(jax.ShapeDtypeStruct((T, D), jnp.float32),
                  jax.ShapeDtypeStruct((T, D), jnp.float32)),
        scratch_types=[
            pltpu.VMEM((bpw,), jnp.int32),
            pltpu.VMEM((bpw,), jnp.int32),
            pltpu.VMEM((bpw, D), jnp.float32),
            pltpu.SemaphoreType.DMA,
        ],
    )
    def gath(yg_hbm, i0_hbm, i1_hbm, o0_hbm, o1_hbm, i0_v, i1_v, rows_v, sem):
        wid = lax.axis_index("s") * 2 + lax.axis_index("c")
        base = wid * bpw
        pltpu.sync_copy(i0_hbm.at[wid], i0_v)
        pltpu.sync_copy(i1_hbm.at[wid], i1_v)
        pltpu.async_copy(yg_hbm.at[i0_v], rows_v, sem).wait()
        pltpu.sync_copy(rows_v, o0_hbm.at[pl.ds(base, bpw)])
        pltpu.async_copy(yg_hbm.at[i1_v], rows_v, sem).wait()
        pltpu.sync_copy(rows_v, o1_hbm.at[pl.ds(base, bpw)])

    return gath(yg, i0, i1)


def _expert_changed(be_ref, nb_ref):
    b = pl.program_id(0)
    e_prev = jnp.where(b == 0, -1, be_ref[jnp.maximum(b - 1, 0)])
    return jnp.logical_and(b < nb_ref[0], be_ref[b] != e_prev)


def _ffn1_kernel(be_ref, nb_ref, xg_ref, w1_ref, b1_ref, h_ref, w1bf_ref):
    b = pl.program_id(0)

    @pl.when(_expert_changed(be_ref, nb_ref))
    def _cast():
        w1bf_ref[...] = w1_ref[0].astype(jnp.bfloat16)

    @pl.when(b < nb_ref[0])
    def _():
        h = jnp.dot(xg_ref[...].astype(jnp.bfloat16), w1bf_ref[...],
                    preferred_element_type=jnp.float32) + b1_ref[0]
        h_ref[...] = jnp.maximum(h, 0.0).astype(jnp.bfloat16)


def _ffn2_kernel(be_ref, nb_ref, h_ref, w2_ref, b2_ref, yg_ref, w2bf_ref):
    b = pl.program_id(0)

    @pl.when(_expert_changed(be_ref, nb_ref))
    def _cast():
        w2bf_ref[...] = w2_ref[0].astype(jnp.bfloat16)

    @pl.when(b < nb_ref[0])
    def _():
        yg_ref[...] = jnp.dot(h_ref[...], w2bf_ref[...],
                              preferred_element_type=jnp.float32) + b2_ref[0]


def _ffn(xg, W1, b1r, W2, b2r, be, nb):
    pbuf, D = xg.shape
    F = W1.shape[2]
    H = pl.pallas_call(
        _ffn1_kernel,
        grid_spec=pltpu.PrefetchScalarGridSpec(
            num_scalar_prefetch=2,
            grid=(MAXB,),
            in_specs=[
                pl.BlockSpec((BLK, D), lambda b, be, nb: (b, 0)),
                pl.BlockSpec((1, D, F), lambda b, be, nb: (be[b], 0, 0)),
                pl.BlockSpec((1, 1, F), lambda b, be, nb: (be[b], 0, 0)),
            ],
            out_specs=pl.BlockSpec((BLK, F), lambda b, be, nb: (b, 0)),
            scratch_shapes=[pltpu.VMEM((D, F), jnp.bfloat16)],
        ),
        out_shape=jax.ShapeDtypeStruct((pbuf, F), jnp.bfloat16),
    )(be, nb, xg, W1, b1r)
    yg = pl.pallas_call(
        _ffn2_kernel,
        grid_spec=pltpu.PrefetchScalarGridSpec(
            num_scalar_prefetch=2,
            grid=(MAXB,),
            in_specs=[
                pl.BlockSpec((BLK, F), lambda b, be, nb: (b, 0)),
                pl.BlockSpec((1, F, D), lambda b, be, nb: (be[b], 0, 0)),
                pl.BlockSpec((1, 1, D), lambda b, be, nb: (be[b], 0, 0)),
            ],
            out_specs=pl.BlockSpec((BLK, D), lambda b, be, nb: (b, 0)),
            scratch_shapes=[pltpu.VMEM((F, D), jnp.bfloat16)],
        ),
        out_shape=jax.ShapeDtypeStruct((pbuf, D), jnp.float32),
    )(be, nb, H, W2, b2r)
    return yg


def _combine_kernel(y0_ref, y1_ref, w0_ref, w1_ref, out_ref):
    out_ref[...] = y0_ref[...] * w0_ref[...] + y1_ref[...] * w1_ref[...]


def _combine(yg0, yg1, w0, w1):
    T, D = yg0.shape
    BT = 512
    return pl.pallas_call(
        _combine_kernel,
        grid=(T // BT,),
        in_specs=[
            pl.BlockSpec((BT, D), lambda i: (i, 0)),
            pl.BlockSpec((BT, D), lambda i: (i, 0)),
            pl.BlockSpec((BT, 1), lambda i: (i, 0)),
            pl.BlockSpec((BT, 1), lambda i: (i, 0)),
        ],
        out_specs=pl.BlockSpec((BT, D), lambda i: (i, 0)),
        out_shape=jax.ShapeDtypeStruct((T, D), jnp.float32),
    )(yg0, yg1, w0, w1)


@jax.jit
def _moe(x2d, Wr, br2, W1, b1r, W2, b2r):
    T, D = x2d.shape
    pbuf = MAXB * BLK
    pos0, pos1, w0, w1, be, nb, xbf = _router(x2d, Wr, br2)
    i0 = pos0.reshape(NW, T // NW)
    i1 = pos1.reshape(NW, T // NW)
    xg = _sc_scatter(x2d, i0, i1, pbuf)
    yg = _ffn(xg, W1, b1r, W2, b2r, be.reshape(MAXB), nb.reshape(1))
    yg0, yg1 = _sc_gather(yg, i0, i1)
    return _combine(yg0, yg1, w0, w1)


def kernel(x, Wr, br, W1, b1, W2, b2):
    B, S, D = x.shape
    x2d = x.reshape(B * S, D)
    out = _moe(x2d, Wr, br.reshape(1, E),
               W1, b1.reshape(E, 1, -1), W2, b2.reshape(E, 1, -1))
    return out.reshape(B, S, D)


# speedup vs baseline: 1.9124x; 1.1645x over previous
import functools

import jax
import jax.numpy as jnp
from jax import lax
from jax.experimental import pallas as pl
import jax.experimental.pallas.tpu as pltpu
from jax.experimental.pallas import tpu_sc as plsc

E = 8
TOPK = 2
BLK = 512
MAXB = 15
NF2 = 4
NW = 32


def _router_kernel(x_ref, wr_ref, br_ref,
                   pos0_ref, pos1_ref, w0_ref, w1_ref, be_ref, nb_ref):
    T = x_ref.shape[0]
    xf = x_ref[...]
    logits = jnp.dot(xf, wr_ref[...],
                     preferred_element_type=jnp.float32) + br_ref[...]
    m = jnp.max(logits, axis=-1, keepdims=True)
    ex = jnp.exp(logits - m)
    probs = ex / jnp.sum(ex, axis=-1, keepdims=True)
    col = lax.broadcasted_iota(jnp.int32, probs.shape, 1)
    big = jnp.int32(E + 1)
    m1 = jnp.max(probs, axis=-1, keepdims=True)
    a1 = jnp.min(jnp.where(probs == m1, col, big), axis=-1, keepdims=True)
    p2 = jnp.where(col == a1, -jnp.inf, probs)
    m2 = jnp.max(p2, axis=-1, keepdims=True)
    a2 = jnp.min(jnp.where(p2 == m2, col, big), axis=-1, keepdims=True)
    denom = m1 + m2
    w0_ref[...] = m1 / denom
    w1_ref[...] = m2 / denom

    Mm = ((col == a1) | (col == a2)).astype(jnp.bfloat16)
    r0 = lax.broadcasted_iota(jnp.int32, (T, T), 0)
    r1 = lax.broadcasted_iota(jnp.int32, (T, T), 1)
    L = (r1 < r0).astype(jnp.bfloat16)
    cnt_excl = jnp.dot(L, Mm, preferred_element_type=jnp.float32)

    counts = jnp.sum(Mm.astype(jnp.float32), axis=0, keepdims=True)
    nb = jnp.floor((counts + (BLK - 1)) / BLK)
    ecol0 = lax.broadcasted_iota(jnp.int32, (E, E), 0)
    ecol1 = lax.broadcasted_iota(jnp.int32, (E, E), 1)
    SU = (ecol0 < ecol1).astype(jnp.float32)
    offs_row = BLK * jnp.dot(nb, SU,
                             preferred_element_type=jnp.float32)

    posf0 = jnp.sum(jnp.where(col == a1, cnt_excl + offs_row, 0.0),
                    axis=-1, keepdims=True)
    posf1 = jnp.sum(jnp.where(col == a2, cnt_excl + offs_row, 0.0),
                    axis=-1, keepdims=True)
    pos0_ref[...] = posf0.astype(jnp.int32)
    pos1_ref[...] = posf1.astype(jnp.int32)

    IdE = (ecol0 == ecol1).astype(jnp.float32)
    SL = (ecol1 < ecol0).astype(jnp.float32)
    nbc = lax.dot_general(IdE, nb, (((1,), (1,)), ((), ())),
                          preferred_element_type=jnp.float32)
    startblk = jnp.dot(SL, nbc, preferred_element_type=jnp.float32)
    total = jnp.sum(nb, axis=-1, keepdims=True)
    bio = lax.broadcasted_iota(jnp.int32, (1, MAXB), 1).astype(jnp.float32)
    bclamp = jnp.minimum(bio, total - 1.0)
    owners = jnp.sum((startblk <= bclamp).astype(jnp.float32),
                     axis=0, keepdims=True)
    be_ref[...] = (owners - 1.0).astype(jnp.int32)
    nb_ref[...] = total.astype(jnp.int32)


def _router(x2d, Wr, br2):
    T, D = x2d.shape
    outs = pl.pallas_call(
        _router_kernel,
        grid=(1,),
        in_specs=[
            pl.BlockSpec((T, D), lambda i: (0, 0)),
            pl.BlockSpec((D, E), lambda i: (0, 0)),
            pl.BlockSpec((1, E), lambda i: (0, 0)),
        ],
        out_specs=[
            pl.BlockSpec((T, 1), lambda i: (0, 0)),
            pl.BlockSpec((T, 1), lambda i: (0, 0)),
            pl.BlockSpec((T, 1), lambda i: (0, 0)),
            pl.BlockSpec((T, 1), lambda i: (0, 0)),
            pl.BlockSpec((1, MAXB), lambda i: (0, 0)),
            pl.BlockSpec((1, 1), lambda i: (0, 0)),
        ],
        out_shape=[
            jax.ShapeDtypeStruct((T, 1), jnp.int32),
            jax.ShapeDtypeStruct((T, 1), jnp.int32),
            jax.ShapeDtypeStruct((T, 1), jnp.float32),
            jax.ShapeDtypeStruct((T, 1), jnp.float32),
            jax.ShapeDtypeStruct((1, MAXB), jnp.int32),
            jax.ShapeDtypeStruct((1, 1), jnp.int32),
        ],
    )(x2d, Wr, br2)
    return outs


def _sc_scatter(x2d, i0, i1, pbuf):
    T, D = x2d.shape
    bpw = T // NW
    mesh = plsc.VectorSubcoreMesh(core_axis_name="c", subcore_axis_name="s")

    @functools.partial(
        pl.kernel, mesh=mesh,
        out_type=jax.ShapeDtypeStruct((pbuf, D), jnp.float32),
        scratch_types=[
            pltpu.VMEM((bpw,), jnp.int32),
            pltpu.VMEM((bpw,), jnp.int32),
            pltpu.VMEM((bpw, D), jnp.float32),
            pltpu.SemaphoreType.DMA,
        ],
    )
    def scat(x_hbm, i0_hbm, i1_hbm, xg_hbm, i0_v, i1_v, rows_v, sem):
        wid = lax.axis_index("s") * 2 + lax.axis_index("c")
        base = wid * bpw
        pltpu.sync_copy(i0_hbm.at[wid], i0_v)
        pltpu.sync_copy(i1_hbm.at[wid], i1_v)
        pltpu.sync_copy(x_hbm.at[pl.ds(base, bpw)], rows_v)
        pltpu.async_copy(rows_v, xg_hbm.at[i0_v], sem).wait()
        pltpu.async_copy(rows_v, xg_hbm.at[i1_v], sem).wait()

    return scat(x2d, i0, i1)


def _sc_gather(yg, i0, i1):
    pbuf, D = yg.shape
    T = i0.shape[0] * i0.shape[1]
    bpw = T // NW
    mesh = plsc.VectorSubcoreMesh(core_axis_name="c", subcore_axis_name="s")

    @functools.partial(
        pl.kernel, mesh=mesh,
        out_type=(jax.ShapeDtypeStruct((T, D), jnp.float32),
                  jax.ShapeDtypeStruct((T, D), jnp.float32)),
        scratch_types=[
            pltpu.VMEM((bpw,), jnp.int32),
            pltpu.VMEM((bpw,), jnp.int32),
            pltpu.VMEM((bpw, D), jnp.float32),
            pltpu.SemaphoreType.DMA,
        ],
    )
    def gath(yg_hbm, i0_hbm, i1_hbm, o0_hbm, o1_hbm, i0_v, i1_v, rows_v, sem):
        wid = lax.axis_index("s") * 2 + lax.axis_index("c")
        base = wid * bpw
        pltpu.sync_copy(i0_hbm.at[wid], i0_v)
        pltpu.sync_copy(i1_hbm.at[wid], i1_v)
        pltpu.async_copy(yg_hbm.at[i0_v], rows_v, sem).wait()
        pltpu.sync_copy(rows_v, o0_hbm.at[pl.ds(base, bpw)])
        pltpu.async_copy(yg_hbm.at[i1_v], rows_v, sem).wait()
        pltpu.sync_copy(rows_v, o1_hbm.at[pl.ds(base, bpw)])

    return gath(yg, i0, i1)


def _ffn_kernel(be_ref, nb_ref, xg_ref, w1_ref, b1_ref, w2_ref, b2_ref,
                yg_ref):
    b = pl.program_id(0)
    j = pl.program_id(1)

    @pl.when(b < nb_ref[0])
    def _():
        xb = xg_ref[...].astype(jnp.bfloat16)
        h = jnp.dot(xb, w1_ref[0].astype(jnp.bfloat16),
                    preferred_element_type=jnp.float32) + b1_ref[0]
        hb = jnp.maximum(h, 0.0).astype(jnp.bfloat16)
        y = jnp.dot(hb, w2_ref[0].astype(jnp.bfloat16),
                    preferred_element_type=jnp.float32)

        @pl.when(j == 0)
        def _init():
            yg_ref[...] = y + b2_ref[0]

        @pl.when(j != 0)
        def _acc():
            yg_ref[...] = yg_ref[...] + y


def _ffn(xg, W1, b1r, W2, b2r, be, nb):
    pbuf, D = xg.shape
    F = W1.shape[2]
    FC = F // NF2

    def _wj(b, j, nb):
        return jnp.where(b < nb[0], j, NF2 - 1)

    yg = pl.pallas_call(
        _ffn_kernel,
        grid_spec=pltpu.PrefetchScalarGridSpec(
            num_scalar_prefetch=2,
            grid=(MAXB, NF2),
            in_specs=[
                pl.BlockSpec((BLK, D), lambda b, j, be, nb: (b, 0)),
                pl.BlockSpec((1, D, FC),
                             lambda b, j, be, nb: (be[b], 0, _wj(b, j, nb))),
                pl.BlockSpec((1, 1, FC),
                             lambda b, j, be, nb: (be[b], 0, _wj(b, j, nb))),
                pl.BlockSpec((1, FC, D),
                             lambda b, j, be, nb: (be[b], _wj(b, j, nb), 0)),
                pl.BlockSpec((1, 1, D), lambda b, j, be, nb: (be[b], 0, 0)),
            ],
            out_specs=pl.BlockSpec((BLK, D), lambda b, j, be, nb: (b, 0)),
        ),
        out_shape=jax.ShapeDtypeStruct((pbuf, D), jnp.float32),
        compiler_params=pltpu.CompilerParams(
            dimension_semantics=("arbitrary", "arbitrary"),
        ),
    )(be, nb, xg, W1, b1r, W2, b2r)
    return yg


def _combine_kernel(y0_ref, y1_ref, w0_ref, w1_ref, out_ref):
    out_ref[...] = y0_ref[...] * w0_ref[...] + y1_ref[...] * w1_ref[...]


def _combine(yg0, yg1, w0, w1):
    T, D = yg0.shape
    BT = 512
    return pl.pallas_call(
        _combine_kernel,
        grid=(T // BT,),
        in_specs=[
            pl.BlockSpec((BT, D), lambda i: (i, 0)),
            pl.BlockSpec((BT, D), lambda i: (i, 0)),
            pl.BlockSpec((BT, 1), lambda i: (i, 0)),
            pl.BlockSpec((BT, 1), lambda i: (i, 0)),
        ],
        out_specs=pl.BlockSpec((BT, D), lambda i: (i, 0)),
        out_shape=jax.ShapeDtypeStruct((T, D), jnp.float32),
    )(yg0, yg1, w0, w1)


@jax.jit
def _moe(x2d, Wr, br2, W1, b1r, W2, b2r):
    T, D = x2d.shape
    pbuf = MAXB * BLK
    pos0, pos1, w0, w1, be, nb = _router(x2d, Wr, br2)
    i0 = pos0.reshape(NW, T // NW)
    i1 = pos1.reshape(NW, T // NW)
    xg = _sc_scatter(x2d, i0, i1, pbuf)
    yg = _ffn(xg, W1, b1r, W2, b2r, be.reshape(MAXB), nb.reshape(1))
    yg0, yg1 = _sc_gather(yg, i0, i1)
    return _combine(yg0, yg1, w0, w1)


def kernel(x, Wr, br, W1, b1, W2, b2):
    B, S, D = x.shape
    x2d = x.reshape(B * S, D)
    out = _moe(x2d, Wr, br.reshape(1, E),
               W1, b1.reshape(E, 1, -1), W2, b2.reshape(E, 1, -1))
    return out.reshape(B, S, D)


# speedup vs baseline: 2.0703x; 1.0825x over previous
import functools

import jax
import jax.numpy as jnp
from jax import lax
from jax.experimental import pallas as pl
import jax.experimental.pallas.tpu as pltpu
from jax.experimental.pallas import tpu_sc as plsc

E = 8
TOPK = 2
BLK = 768
MAXB = 13
NF2 = 4
NW = 32


def _router_kernel(x_ref, wr_ref, br_ref,
                   pos0_ref, pos1_ref, w0_ref, w1_ref, be_ref, nb_ref):
    T = x_ref.shape[0]
    xf = x_ref[...]
    logits = jnp.dot(xf, wr_ref[...],
                     preferred_element_type=jnp.float32) + br_ref[...]
    m = jnp.max(logits, axis=-1, keepdims=True)
    ex = jnp.exp(logits - m)
    probs = ex / jnp.sum(ex, axis=-1, keepdims=True)
    col = lax.broadcasted_iota(jnp.int32, probs.shape, 1)
    big = jnp.int32(E + 1)
    m1 = jnp.max(probs, axis=-1, keepdims=True)
    a1 = jnp.min(jnp.where(probs == m1, col, big), axis=-1, keepdims=True)
    p2 = jnp.where(col == a1, -jnp.inf, probs)
    m2 = jnp.max(p2, axis=-1, keepdims=True)
    a2 = jnp.min(jnp.where(p2 == m2, col, big), axis=-1, keepdims=True)
    denom = m1 + m2
    w0_ref[...] = m1 / denom
    w1_ref[...] = m2 / denom

    Mm = ((col == a1) | (col == a2)).astype(jnp.bfloat16)
    r0 = lax.broadcasted_iota(jnp.int32, (T, T), 0)
    r1 = lax.broadcasted_iota(jnp.int32, (T, T), 1)
    L = (r1 < r0).astype(jnp.bfloat16)
    cnt_excl = jnp.dot(L, Mm, preferred_element_type=jnp.float32)

    counts = jnp.sum(Mm.astype(jnp.float32), axis=0, keepdims=True)
    nb = jnp.floor((counts + (BLK - 1)) / BLK)
    ecol0 = lax.broadcasted_iota(jnp.int32, (E, E), 0)
    ecol1 = lax.broadcasted_iota(jnp.int32, (E, E), 1)
    SU = (ecol0 < ecol1).astype(jnp.float32)
    offs_row = BLK * jnp.dot(nb, SU,
                             preferred_element_type=jnp.float32)

    posf0 = jnp.sum(jnp.where(col == a1, cnt_excl + offs_row, 0.0),
                    axis=-1, keepdims=True)
    posf1 = jnp.sum(jnp.where(col == a2, cnt_excl + offs_row, 0.0),
                    axis=-1, keepdims=True)
    pos0_ref[...] = posf0.astype(jnp.int32)
    pos1_ref[...] = posf1.astype(jnp.int32)

    IdE = (ecol0 == ecol1).astype(jnp.float32)
    SL = (ecol1 < ecol0).astype(jnp.float32)
    nbc = lax.dot_general(IdE, nb, (((1,), (1,)), ((), ())),
                          preferred_element_type=jnp.float32)
    startblk = jnp.dot(SL, nbc, preferred_element_type=jnp.float32)
    total = jnp.sum(nb, axis=-1, keepdims=True)
    bio = lax.broadcasted_iota(jnp.int32, (1, MAXB), 1).astype(jnp.float32)
    bclamp = jnp.minimum(bio, total - 1.0)
    owners = jnp.sum((startblk <= bclamp).astype(jnp.float32),
                     axis=0, keepdims=True)
    be_ref[...] = (owners - 1.0).astype(jnp.int32)
    nb_ref[...] = total.astype(jnp.int32)


def _router(x2d, Wr, br2):
    T, D = x2d.shape
    outs = pl.pallas_call(
        _router_kernel,
        grid=(1,),
        in_specs=[
            pl.BlockSpec((T, D), lambda i: (0, 0)),
            pl.BlockSpec((D, E), lambda i: (0, 0)),
            pl.BlockSpec((1, E), lambda i: (0, 0)),
        ],
        out_specs=[
            pl.BlockSpec((T, 1), lambda i: (0, 0)),
            pl.BlockSpec((T, 1), lambda i: (0, 0)),
            pl.BlockSpec((T, 1), lambda i: (0, 0)),
            pl.BlockSpec((T, 1), lambda i: (0, 0)),
            pl.BlockSpec((1, MAXB), lambda i: (0, 0)),
            pl.BlockSpec((1, 1), lambda i: (0, 0)),
        ],
        out_shape=[
            jax.ShapeDtypeStruct((T, 1), jnp.int32),
            jax.ShapeDtypeStruct((T, 1), jnp.int32),
            jax.ShapeDtypeStruct((T, 1), jnp.float32),
            jax.ShapeDtypeStruct((T, 1), jnp.float32),
            jax.ShapeDtypeStruct((1, MAXB), jnp.int32),
            jax.ShapeDtypeStruct((1, 1), jnp.int32),
        ],
    )(x2d, Wr, br2)
    return outs


def _sc_scatter(x2d, i0, i1, pbuf):
    T, D = x2d.shape
    bpw = T // NW
    mesh = plsc.VectorSubcoreMesh(core_axis_name="c", subcore_axis_name="s")

    @functools.partial(
        pl.kernel, mesh=mesh,
        out_type=jax.ShapeDtypeStruct((pbuf, D), jnp.float32),
        scratch_types=[
            pltpu.VMEM((bpw,), jnp.int32),
            pltpu.VMEM((bpw,), jnp.int32),
            pltpu.VMEM((bpw, D), jnp.float32),
            pltpu.SemaphoreType.DMA,
        ],
    )
    def scat(x_hbm, i0_hbm, i1_hbm, xg_hbm, i0_v, i1_v, rows_v, sem):
        wid = lax.axis_index("s") * 2 + lax.axis_index("c")
        base = wid * bpw
        pltpu.sync_copy(i0_hbm.at[wid], i0_v)
        pltpu.sync_copy(i1_hbm.at[wid], i1_v)
        pltpu.sync_copy(x_hbm.at[pl.ds(base, bpw)], rows_v)
        pltpu.async_copy(rows_v, xg_hbm.at[i0_v], sem).wait()
        pltpu.async_copy(rows_v, xg_hbm.at[i1_v], sem).wait()

    return scat(x2d, i0, i1)


def _sc_gather(yg, i0, i1):
    pbuf, D = yg.shape
    T = i0.shape[0] * i0.shape[1]
    bpw = T // NW
    mesh = plsc.VectorSubcoreMesh(core_axis_name="c", subcore_axis_name="s")

    @functools.partial(
        pl.kernel, mesh=mesh,
        out_type=(jax.ShapeDtypeStruct((T, D), jnp.float32),
                  jax.ShapeDtypeStruct((T, D), jnp.float32)),
        scratch_types=[
            pltpu.VMEM((bpw,), jnp.int32),
            pltpu.VMEM((bpw,), jnp.int32),
            pltpu.VMEM((bpw, D), jnp.float32),
            pltpu.SemaphoreType.DMA,
        ],
    )
    def gath(yg_hbm, i0_hbm, i1_hbm, o0_hbm, o1_hbm, i0_v, i1_v, rows_v, sem):
        wid = lax.axis_index("s") * 2 + lax.axis_index("c")
        base = wid * bpw
        pltpu.sync_copy(i0_hbm.at[wid], i0_v)
        pltpu.sync_copy(i1_hbm.at[wid], i1_v)
        pltpu.async_copy(yg_hbm.at[i0_v], rows_v, sem).wait()
        pltpu.sync_copy(rows_v, o0_hbm.at[pl.ds(base, bpw)])
        pltpu.async_copy(yg_hbm.at[i1_v], rows_v, sem).wait()
        pltpu.sync_copy(rows_v, o1_hbm.at[pl.ds(base, bpw)])

    return gath(yg, i0, i1)


def _ffn_kernel(be_ref, nb_ref, xg_ref, w1_ref, b1_ref, w2_ref, b2_ref,
                yg_ref):
    b = pl.program_id(0)
    j = pl.program_id(1)

    @pl.when(b < nb_ref[0])
    def _():
        xb = xg_ref[...].astype(jnp.bfloat16)
        h = jnp.dot(xb, w1_ref[0].astype(jnp.bfloat16),
                    preferred_element_type=jnp.float32) + b1_ref[0]
        hb = jnp.maximum(h, 0.0).astype(jnp.bfloat16)
        y = jnp.dot(hb, w2_ref[0].astype(jnp.bfloat16),
                    preferred_element_type=jnp.float32)

        @pl.when(j == 0)
        def _init():
            yg_ref[...] = y + b2_ref[0]

        @pl.when(j != 0)
        def _acc():
            yg_ref[...] = yg_ref[...] + y


def _ffn(xg, W1, b1r, W2, b2r, be, nb):
    pbuf, D = xg.shape
    F = W1.shape[2]
    FC = F // NF2

    def _wj(b, j, nb):
        return jnp.where(b < nb[0], j, NF2 - 1)

    yg = pl.pallas_call(
        _ffn_kernel,
        grid_spec=pltpu.PrefetchScalarGridSpec(
            num_scalar_prefetch=2,
            grid=(MAXB, NF2),
            in_specs=[
                pl.BlockSpec((BLK, D), lambda b, j, be, nb: (b, 0)),
                pl.BlockSpec((1, D, FC),
                             lambda b, j, be, nb: (be[b], 0, _wj(b, j, nb))),
                pl.BlockSpec((1, 1, FC),
                             lambda b, j, be, nb: (be[b], 0, _wj(b, j, nb))),
                pl.BlockSpec((1, FC, D),
                             lambda b, j, be, nb: (be[b], _wj(b, j, nb), 0)),
                pl.BlockSpec((1, 1, D), lambda b, j, be, nb: (be[b], 0, 0)),
            ],
            out_specs=pl.BlockSpec((BLK, D), lambda b, j, be, nb: (b, 0)),
        ),
        out_shape=jax.ShapeDtypeStruct((pbuf, D), jnp.float32),
        compiler_params=pltpu.CompilerParams(
            dimension_semantics=("arbitrary", "arbitrary"),
        ),
    )(be, nb, xg, W1, b1r, W2, b2r)
    return yg


def _combine_kernel(y0_ref, y1_ref, w0_ref, w1_ref, out_ref):
    out_ref[...] = y0_ref[...] * w0_ref[...] + y1_ref[...] * w1_ref[...]


def _combine(yg0, yg1, w0, w1):
    T, D = yg0.shape
    BT = 512
    return pl.pallas_call(
        _combine_kernel,
        grid=(T // BT,),
        in_specs=[
            pl.BlockSpec((BT, D), lambda i: (i, 0)),
            pl.BlockSpec((BT, D), lambda i: (i, 0)),
            pl.BlockSpec((BT, 1), lambda i: (i, 0)),
            pl.BlockSpec((BT, 1), lambda i: (i, 0)),
        ],
        out_specs=pl.BlockSpec((BT, D), lambda i: (i, 0)),
        out_shape=jax.ShapeDtypeStruct((T, D), jnp.float32),
    )(yg0, yg1, w0, w1)


@jax.jit
def _moe(x2d, Wr, br2, W1, b1r, W2, b2r):
    T, D = x2d.shape
    pbuf = MAXB * BLK
    pos0, pos1, w0, w1, be, nb = _router(x2d, Wr, br2)
    i0 = pos0.reshape(NW, T // NW)
    i1 = pos1.reshape(NW, T // NW)
    xg = _sc_scatter(x2d, i0, i1, pbuf)
    yg = _ffn(xg, W1, b1r, W2, b2r, be.reshape(MAXB), nb.reshape(1))
    yg0, yg1 = _sc_gather(yg, i0, i1)
    return _combine(yg0, yg1, w0, w1)


def kernel(x, Wr, br, W1, b1, W2, b2):
    B, S, D = x.shape
    x2d = x.reshape(B * S, D)
    out = _moe(x2d, Wr, br.reshape(1, E),
               W1, b1.reshape(E, 1, -1), W2, b2.reshape(E, 1, -1))
    return out.reshape(B, S, D)


# speedup vs baseline: 2.1978x; 1.0616x over previous
import functools

import jax
import jax.numpy as jnp
from jax import lax
from jax.experimental import pallas as pl
import jax.experimental.pallas.tpu as pltpu
from jax.experimental.pallas import tpu_sc as plsc

E = 8
TOPK = 2
BLK = 768
MAXB = 13
NF2 = 2
NW = 32


def _router_kernel(x_ref, wr_ref, br_ref,
                   pos0_ref, pos1_ref, w0_ref, w1_ref, be_ref, nb_ref):
    T = x_ref.shape[0]
    xf = x_ref[...]
    logits = jnp.dot(xf, wr_ref[...],
                     preferred_element_type=jnp.float32) + br_ref[...]
    m = jnp.max(logits, axis=-1, keepdims=True)
    ex = jnp.exp(logits - m)
    probs = ex / jnp.sum(ex, axis=-1, keepdims=True)
    col = lax.broadcasted_iota(jnp.int32, probs.shape, 1)
    big = jnp.int32(E + 1)
    m1 = jnp.max(probs, axis=-1, keepdims=True)
    a1 = jnp.min(jnp.where(probs == m1, col, big), axis=-1, keepdims=True)
    p2 = jnp.where(col == a1, -jnp.inf, probs)
    m2 = jnp.max(p2, axis=-1, keepdims=True)
    a2 = jnp.min(jnp.where(p2 == m2, col, big), axis=-1, keepdims=True)
    denom = m1 + m2
    w0_ref[...] = m1 / denom
    w1_ref[...] = m2 / denom

    Mm = ((col == a1) | (col == a2)).astype(jnp.bfloat16)
    r0 = lax.broadcasted_iota(jnp.int32, (T, T), 0)
    r1 = lax.broadcasted_iota(jnp.int32, (T, T), 1)
    L = (r1 < r0).astype(jnp.bfloat16)
    cnt_excl = jnp.dot(L, Mm, preferred_element_type=jnp.float32)

    counts = jnp.sum(Mm.astype(jnp.float32), axis=0, keepdims=True)
    nb = jnp.floor((counts + (BLK - 1)) / BLK)
    ecol0 = lax.broadcasted_iota(jnp.int32, (E, E), 0)
    ecol1 = lax.broadcasted_iota(jnp.int32, (E, E), 1)
    SU = (ecol0 < ecol1).astype(jnp.float32)
    offs_row = BLK * jnp.dot(nb, SU,
                             preferred_element_type=jnp.float32)

    posf0 = jnp.sum(jnp.where(col == a1, cnt_excl + offs_row, 0.0),
                    axis=-1, keepdims=True)
    posf1 = jnp.sum(jnp.where(col == a2, cnt_excl + offs_row, 0.0),
                    axis=-1, keepdims=True)
    pos0_ref[...] = posf0.astype(jnp.int32)
    pos1_ref[...] = posf1.astype(jnp.int32)

    IdE = (ecol0 == ecol1).astype(jnp.float32)
    SL = (ecol1 < ecol0).astype(jnp.float32)
    nbc = lax.dot_general(IdE, nb, (((1,), (1,)), ((), ())),
                          preferred_element_type=jnp.float32)
    startblk = jnp.dot(SL, nbc, preferred_element_type=jnp.float32)
    total = jnp.sum(nb, axis=-1, keepdims=True)
    bio = lax.broadcasted_iota(jnp.int32, (1, MAXB), 1).astype(jnp.float32)
    bclamp = jnp.minimum(bio, total - 1.0)
    owners = jnp.sum((startblk <= bclamp).astype(jnp.float32),
                     axis=0, keepdims=True)
    be_ref[...] = (owners - 1.0).astype(jnp.int32)
    nb_ref[...] = total.astype(jnp.int32)


def _router(x2d, Wr, br2):
    T, D = x2d.shape
    outs = pl.pallas_call(
        _router_kernel,
        grid=(1,),
        in_specs=[
            pl.BlockSpec((T, D), lambda i: (0, 0)),
            pl.BlockSpec((D, E), lambda i: (0, 0)),
            pl.BlockSpec((1, E), lambda i: (0, 0)),
        ],
        out_specs=[
            pl.BlockSpec((T, 1), lambda i: (0, 0)),
            pl.BlockSpec((T, 1), lambda i: (0, 0)),
            pl.BlockSpec((T, 1), lambda i: (0, 0)),
            pl.BlockSpec((T, 1), lambda i: (0, 0)),
            pl.BlockSpec((1, MAXB), lambda i: (0, 0)),
            pl.BlockSpec((1, 1), lambda i: (0, 0)),
        ],
        out_shape=[
            jax.ShapeDtypeStruct((T, 1), jnp.int32),
            jax.ShapeDtypeStruct((T, 1), jnp.int32),
            jax.ShapeDtypeStruct((T, 1), jnp.float32),
            jax.ShapeDtypeStruct((T, 1), jnp.float32),
            jax.ShapeDtypeStruct((1, MAXB), jnp.int32),
            jax.ShapeDtypeStruct((1, 1), jnp.int32),
        ],
    )(x2d, Wr, br2)
    return outs


def _sc_scatter(x2d, i0, i1, pbuf):
    T, D = x2d.shape
    bpw = T // NW
    mesh = plsc.VectorSubcoreMesh(core_axis_name="c", subcore_axis_name="s")

    @functools.partial(
        pl.kernel, mesh=mesh,
        out_type=jax.ShapeDtypeStruct((pbuf, D), jnp.float32),
        scratch_types=[
            pltpu.VMEM((bpw,), jnp.int32),
            pltpu.VMEM((bpw,), jnp.int32),
            pltpu.VMEM((bpw, D), jnp.float32),
            pltpu.SemaphoreType.DMA,
        ],
    )
    def scat(x_hbm, i0_hbm, i1_hbm, xg_hbm, i0_v, i1_v, rows_v, sem):
        wid = lax.axis_index("s") * 2 + lax.axis_index("c")
        base = wid * bpw
        pltpu.sync_copy(i0_hbm.at[wid], i0_v)
        pltpu.sync_copy(i1_hbm.at[wid], i1_v)
        pltpu.sync_copy(x_hbm.at[pl.ds(base, bpw)], rows_v)
        pltpu.async_copy(rows_v, xg_hbm.at[i0_v], sem).wait()
        pltpu.async_copy(rows_v, xg_hbm.at[i1_v], sem).wait()

    return scat(x2d, i0, i1)


def _sc_gather(yg, i0, i1):
    pbuf, D = yg.shape
    T = i0.shape[0] * i0.shape[1]
    bpw = T // NW
    mesh = plsc.VectorSubcoreMesh(core_axis_name="c", subcore_axis_name="s")

    @functools.partial(
        pl.kernel, mesh=mesh,
        out_type=(jax.ShapeDtypeStruct((T, D), jnp.float32),
                  jax.ShapeDtypeStruct((T, D), jnp.float32)),
        scratch_types=[
            pltpu.VMEM((bpw,), jnp.int32),
            pltpu.VMEM((bpw,), jnp.int32),
            pltpu.VMEM((bpw, D), jnp.float32),
            pltpu.SemaphoreType.DMA,
        ],
    )
    def gath(yg_hbm, i0_hbm, i1_hbm, o0_hbm, o1_hbm, i0_v, i1_v, rows_v, sem):
        wid = lax.axis_index("s") * 2 + lax.axis_index("c")
        base = wid * bpw
        pltpu.sync_copy(i0_hbm.at[wid], i0_v)
        pltpu.sync_copy(i1_hbm.at[wid], i1_v)
        pltpu.async_copy(yg_hbm.at[i0_v], rows_v, sem).wait()
        pltpu.sync_copy(rows_v, o0_hbm.at[pl.ds(base, bpw)])
        pltpu.async_copy(yg_hbm.at[i1_v], rows_v, sem).wait()
        pltpu.sync_copy(rows_v, o1_hbm.at[pl.ds(base, bpw)])

    return gath(yg, i0, i1)


def _ffn_kernel(be_ref, nb_ref, xg_ref, w1_ref, b1_ref, w2_ref, b2_ref,
                yg_ref):
    b = pl.program_id(0)
    j = pl.program_id(1)

    @pl.when(b < nb_ref[0])
    def _():
        xb = xg_ref[...].astype(jnp.bfloat16)
        h = jnp.dot(xb, w1_ref[0].astype(jnp.bfloat16),
                    preferred_element_type=jnp.float32) + b1_ref[0]
        hb = jnp.maximum(h, 0.0).astype(jnp.bfloat16)
        y = jnp.dot(hb, w2_ref[0].astype(jnp.bfloat16),
                    preferred_element_type=jnp.float32)

        @pl.when(j == 0)
        def _init():
            yg_ref[...] = y + b2_ref[0]

        @pl.when(j != 0)
        def _acc():
            yg_ref[...] = yg_ref[...] + y


def _ffn(xg, W1, b1r, W2, b2r, be, nb):
    pbuf, D = xg.shape
    F = W1.shape[2]
    FC = F // NF2

    def _wj(b, j, nb):
        return jnp.where(b < nb[0], j, NF2 - 1)

    yg = pl.pallas_call(
        _ffn_kernel,
        grid_spec=pltpu.PrefetchScalarGridSpec(
            num_scalar_prefetch=2,
            grid=(MAXB, NF2),
            in_specs=[
                pl.BlockSpec((BLK, D), lambda b, j, be, nb: (b, 0)),
                pl.BlockSpec((1, D, FC),
                             lambda b, j, be, nb: (be[b], 0, _wj(b, j, nb))),
                pl.BlockSpec((1, 1, FC),
                             lambda b, j, be, nb: (be[b], 0, _wj(b, j, nb))),
                pl.BlockSpec((1, FC, D),
                             lambda b, j, be, nb: (be[b], _wj(b, j, nb), 0)),
                pl.BlockSpec((1, 1, D), lambda b, j, be, nb: (be[b], 0, 0)),
            ],
            out_specs=pl.BlockSpec((BLK, D), lambda b, j, be, nb: (b, 0)),
        ),
        out_shape=jax.ShapeDtypeStruct((pbuf, D), jnp.float32),
        compiler_params=pltpu.CompilerParams(
            dimension_semantics=("arbitrary", "arbitrary"),
        ),
    )(be, nb, xg, W1, b1r, W2, b2r)
    return yg


def _combine_kernel(y0_ref, y1_ref, w0_ref, w1_ref, out_ref):
    out_ref[...] = y0_ref[...] * w0_ref[...] + y1_ref[...] * w1_ref[...]


def _combine(yg0, yg1, w0, w1):
    T, D = yg0.shape
    BT = 512
    return pl.pallas_call(
        _combine_kernel,
        grid=(T // BT,),
        in_specs=[
            pl.BlockSpec((BT, D), lambda i: (i, 0)),
            pl.BlockSpec((BT, D), lambda i: (i, 0)),
            pl.BlockSpec((BT, 1), lambda i: (i, 0)),
            pl.BlockSpec((BT, 1), lambda i: (i, 0)),
        ],
        out_specs=pl.BlockSpec((BT, D), lambda i: (i, 0)),
        out_shape=jax.ShapeDtypeStruct((T, D), jnp.float32),
    )(yg0, yg1, w0, w1)


@jax.jit
def _moe(x2d, Wr, br2, W1, b1r, W2, b2r):
    T, D = x2d.shape
    pbuf = MAXB * BLK
    pos0, pos1, w0, w1, be, nb = _router(x2d, Wr, br2)
    i0 = pos0.reshape(NW, T // NW)
    i1 = pos1.reshape(NW, T // NW)
    xg = _sc_scatter(x2d, i0, i1, pbuf)
    yg = _ffn(xg, W1, b1r, W2, b2r, be.reshape(MAXB), nb.reshape(1))
    yg0, yg1 = _sc_gather(yg, i0, i1)
    return _combine(yg0, yg1, w0, w1)


def kernel(x, Wr, br, W1, b1, W2, b2):
    B, S, D = x.shape
    x2d = x.reshape(B * S, D)
    out = _moe(x2d, Wr, br.reshape(1, E),
               W1, b1.reshape(E, 1, -1), W2, b2.reshape(E, 1, -1))
    return out.reshape(B, S, D)


# speedup vs baseline: 2.4944x; 1.1350x over previous
import functools

import jax
import jax.numpy as jnp
from jax import lax
from jax.experimental import pallas as pl
import jax.experimental.pallas.tpu as pltpu
from jax.experimental.pallas import tpu_sc as plsc

E = 8
TOPK = 2
BLK = 640
MAXB = 14
NF2 = 2
NW = 32


def _router_kernel(x_ref, wr_ref, br_ref,
                   pos0_ref, pos1_ref, w0_ref, w1_ref, be_ref, nb_ref):
    T = x_ref.shape[0]
    xf = x_ref[...]
    logits = jnp.dot(xf, wr_ref[...],
                     preferred_element_type=jnp.float32) + br_ref[...]
    m = jnp.max(logits, axis=-1, keepdims=True)
    ex = jnp.exp(logits - m)
    probs = ex / jnp.sum(ex, axis=-1, keepdims=True)
    col = lax.broadcasted_iota(jnp.int32, probs.shape, 1)
    big = jnp.int32(E + 1)
    m1 = jnp.max(probs, axis=-1, keepdims=True)
    a1 = jnp.min(jnp.where(probs == m1, col, big), axis=-1, keepdims=True)
    p2 = jnp.where(col == a1, -jnp.inf, probs)
    m2 = jnp.max(p2, axis=-1, keepdims=True)
    a2 = jnp.min(jnp.where(p2 == m2, col, big), axis=-1, keepdims=True)
    denom = m1 + m2
    w0_ref[...] = m1 / denom
    w1_ref[...] = m2 / denom

    Mm = ((col == a1) | (col == a2)).astype(jnp.bfloat16)
    r0 = lax.broadcasted_iota(jnp.int32, (T, T), 0)
    r1 = lax.broadcasted_iota(jnp.int32, (T, T), 1)
    L = (r1 < r0).astype(jnp.bfloat16)
    cnt_excl = jnp.dot(L, Mm, preferred_element_type=jnp.float32)

    counts = jnp.sum(Mm.astype(jnp.float32), axis=0, keepdims=True)
    nb = jnp.floor((counts + (BLK - 1)) / BLK)
    ecol0 = lax.broadcasted_iota(jnp.int32, (E, E), 0)
    ecol1 = lax.broadcasted_iota(jnp.int32, (E, E), 1)
    SU = (ecol0 < ecol1).astype(jnp.float32)
    offs_row = BLK * jnp.dot(nb, SU,
                             preferred_element_type=jnp.float32)

    posf0 = jnp.sum(jnp.where(col == a1, cnt_excl + offs_row, 0.0),
                    axis=-1, keepdims=True)
    posf1 = jnp.sum(jnp.where(col == a2, cnt_excl + offs_row, 0.0),
                    axis=-1, keepdims=True)
    pos0_ref[...] = posf0.astype(jnp.int32)
    pos1_ref[...] = posf1.astype(jnp.int32)

    IdE = (ecol0 == ecol1).astype(jnp.float32)
    SL = (ecol1 < ecol0).astype(jnp.float32)
    nbc = lax.dot_general(IdE, nb, (((1,), (1,)), ((), ())),
                          preferred_element_type=jnp.float32)
    startblk = jnp.dot(SL, nbc, preferred_element_type=jnp.float32)
    total = jnp.sum(nb, axis=-1, keepdims=True)
    bio = lax.broadcasted_iota(jnp.int32, (1, MAXB), 1).astype(jnp.float32)
    bclamp = jnp.minimum(bio, total - 1.0)
    owners = jnp.sum((startblk <= bclamp).astype(jnp.float32),
                     axis=0, keepdims=True)
    be_ref[...] = (owners - 1.0).astype(jnp.int32)
    nb_ref[...] = total.astype(jnp.int32)


def _router(x2d, Wr, br2):
    T, D = x2d.shape
    outs = pl.pallas_call(
        _router_kernel,
        grid=(1,),
        in_specs=[
            pl.BlockSpec((T, D), lambda i: (0, 0)),
            pl.BlockSpec((D, E), lambda i: (0, 0)),
            pl.BlockSpec((1, E), lambda i: (0, 0)),
        ],
        out_specs=[
            pl.BlockSpec((T, 1), lambda i: (0, 0)),
            pl.BlockSpec((T, 1), lambda i: (0, 0)),
            pl.BlockSpec((T, 1), lambda i: (0, 0)),
            pl.BlockSpec((T, 1), lambda i: (0, 0)),
            pl.BlockSpec((1, MAXB), lambda i: (0, 0)),
            pl.BlockSpec((1, 1), lambda i: (0, 0)),
        ],
        out_shape=[
            jax.ShapeDtypeStruct((T, 1), jnp.int32),
            jax.ShapeDtypeStruct((T, 1), jnp.int32),
            jax.ShapeDtypeStruct((T, 1), jnp.float32),
            jax.ShapeDtypeStruct((T, 1), jnp.float32),
            jax.ShapeDtypeStruct((1, MAXB), jnp.int32),
            jax.ShapeDtypeStruct((1, 1), jnp.int32),
        ],
    )(x2d, Wr, br2)
    return outs


def _sc_scatter(x2d, i0, i1, pbuf):
    T, D = x2d.shape
    bpw = T // NW
    mesh = plsc.VectorSubcoreMesh(core_axis_name="c", subcore_axis_name="s")

    @functools.partial(
        pl.kernel, mesh=mesh,
        out_type=jax.ShapeDtypeStruct((pbuf, D), jnp.float32),
        scratch_types=[
            pltpu.VMEM((bpw,), jnp.int32),
            pltpu.VMEM((bpw,), jnp.int32),
            pltpu.VMEM((bpw, D), jnp.float32),
            pltpu.SemaphoreType.DMA,
        ],
    )
    def scat(x_hbm, i0_hbm, i1_hbm, xg_hbm, i0_v, i1_v, rows_v, sem):
        wid = lax.axis_index("s") * 2 + lax.axis_index("c")
        base = wid * bpw
        pltpu.sync_copy(i0_hbm.at[wid], i0_v)
        pltpu.sync_copy(i1_hbm.at[wid], i1_v)
        pltpu.sync_copy(x_hbm.at[pl.ds(base, bpw)], rows_v)
        pltpu.async_copy(rows_v, xg_hbm.at[i0_v], sem).wait()
        pltpu.async_copy(rows_v, xg_hbm.at[i1_v], sem).wait()

    return scat(x2d, i0, i1)


def _sc_gather(yg, i0, i1):
    pbuf, D = yg.shape
    T = i0.shape[0] * i0.shape[1]
    bpw = T // NW
    mesh = plsc.VectorSubcoreMesh(core_axis_name="c", subcore_axis_name="s")

    @functools.partial(
        pl.kernel, mesh=mesh,
        out_type=(jax.ShapeDtypeStruct((T, D), jnp.float32),
                  jax.ShapeDtypeStruct((T, D), jnp.float32)),
        scratch_types=[
            pltpu.VMEM((bpw,), jnp.int32),
            pltpu.VMEM((bpw,), jnp.int32),
            pltpu.VMEM((bpw, D), jnp.float32),
            pltpu.SemaphoreType.DMA,
        ],
    )
    def gath(yg_hbm, i0_hbm, i1_hbm, o0_hbm, o1_hbm, i0_v, i1_v, rows_v, sem):
        wid = lax.axis_index("s") * 2 + lax.axis_index("c")
        base = wid * bpw
        pltpu.sync_copy(i0_hbm.at[wid], i0_v)
        pltpu.sync_copy(i1_hbm.at[wid], i1_v)
        pltpu.async_copy(yg_hbm.at[i0_v], rows_v, sem).wait()
        pltpu.sync_copy(rows_v, o0_hbm.at[pl.ds(base, bpw)])
        pltpu.async_copy(yg_hbm.at[i1_v], rows_v, sem).wait()
        pltpu.sync_copy(rows_v, o1_hbm.at[pl.ds(base, bpw)])

    return gath(yg, i0, i1)


def _ffn_kernel(be_ref, nb_ref, xg_ref, w1_ref, b1_ref, w2_ref, b2_ref,
                yg_ref):
    b = pl.program_id(0)
    j = pl.program_id(1)

    @pl.when(b < nb_ref[0])
    def _():
        xb = xg_ref[...].astype(jnp.bfloat16)
        h = jnp.dot(xb, w1_ref[0].astype(jnp.bfloat16),
                    preferred_element_type=jnp.float32) + b1_ref[0]
        hb = jnp.maximum(h, 0.0).astype(jnp.bfloat16)
        y = jnp.dot(hb, w2_ref[0].astype(jnp.bfloat16),
                    preferred_element_type=jnp.float32)

        @pl.when(j == 0)
        def _init():
            yg_ref[...] = y + b2_ref[0]

        @pl.when(j != 0)
        def _acc():
            yg_ref[...] = yg_ref[...] + y


def _ffn(xg, W1, b1r, W2, b2r, be, nb):
    pbuf, D = xg.shape
    F = W1.shape[2]
    FC = F // NF2

    def _wj(b, j, nb):
        return jnp.where(b < nb[0], j, NF2 - 1)

    def _bc(b, nb):
        return jnp.minimum(b, nb[0] - 1)

    yg = pl.pallas_call(
        _ffn_kernel,
        grid_spec=pltpu.PrefetchScalarGridSpec(
            num_scalar_prefetch=2,
            grid=(MAXB, NF2),
            in_specs=[
                pl.BlockSpec((BLK, D), lambda b, j, be, nb: (_bc(b, nb), 0)),
                pl.BlockSpec((1, D, FC),
                             lambda b, j, be, nb: (be[b], 0, _wj(b, j, nb))),
                pl.BlockSpec((1, 1, FC),
                             lambda b, j, be, nb: (be[b], 0, _wj(b, j, nb))),
                pl.BlockSpec((1, FC, D),
                             lambda b, j, be, nb: (be[b], _wj(b, j, nb), 0)),
                pl.BlockSpec((1, 1, D), lambda b, j, be, nb: (be[b], 0, 0)),
            ],
            out_specs=pl.BlockSpec((BLK, D),
                                   lambda b, j, be, nb: (_bc(b, nb), 0)),
        ),
        out_shape=jax.ShapeDtypeStruct((pbuf, D), jnp.float32),
        compiler_params=pltpu.CompilerParams(
            dimension_semantics=("arbitrary", "arbitrary"),
        ),
    )(be, nb, xg, W1, b1r, W2, b2r)
    return yg


def _combine_kernel(y0_ref, y1_ref, w0_ref, w1_ref, out_ref):
    out_ref[...] = y0_ref[...] * w0_ref[...] + y1_ref[...] * w1_ref[...]


def _combine(yg0, yg1, w0, w1):
    T, D = yg0.shape
    BT = 512
    return pl.pallas_call(
        _combine_kernel,
        grid=(T // BT,),
        in_specs=[
            pl.BlockSpec((BT, D), lambda i: (i, 0)),
            pl.BlockSpec((BT, D), lambda i: (i, 0)),
            pl.BlockSpec((BT, 1), lambda i: (i, 0)),
            pl.BlockSpec((BT, 1), lambda i: (i, 0)),
        ],
        out_specs=pl.BlockSpec((BT, D), lambda i: (i, 0)),
        out_shape=jax.ShapeDtypeStruct((T, D), jnp.float32),
    )(yg0, yg1, w0, w1)


@jax.jit
def _moe(x2d, Wr, br2, W1, b1r, W2, b2r):
    T, D = x2d.shape
    pbuf = MAXB * BLK
    pos0, pos1, w0, w1, be, nb = _router(x2d, Wr, br2)
    i0 = pos0.reshape(NW, T // NW)
    i1 = pos1.reshape(NW, T // NW)
    xg = _sc_scatter(x2d, i0, i1, pbuf)
    yg = _ffn(xg, W1, b1r, W2, b2r, be.reshape(MAXB), nb.reshape(1))
    yg0, yg1 = _sc_gather(yg, i0, i1)
    return _combine(yg0, yg1, w0, w1)


def kernel(x, Wr, br, W1, b1, W2, b2):
    B, S, D = x.shape
    x2d = x.reshape(B * S, D)
    out = _moe(x2d, Wr, br.reshape(1, E),
               W1, b1.reshape(E, 1, -1), W2, b2.reshape(E, 1, -1))
    return out.reshape(B, S, D)


# speedup vs baseline: 2.5552x; 1.0244x over previous
import functools

import jax
import jax.numpy as jnp
from jax import lax
from jax.experimental import pallas as pl
import jax.experimental.pallas.tpu as pltpu
from jax.experimental.pallas import tpu_sc as plsc

E = 8
TOPK = 2
BLK = 576
MAXB = 15
NF2 = 2
NW = 32


def _router_kernel(x_ref, wr_ref, br_ref,
                   pos0_ref, pos1_ref, w0_ref, w1_ref, be_ref, nb_ref):
    T = x_ref.shape[0]
    xf = x_ref[...]
    logits = jnp.dot(xf, wr_ref[...],
                     preferred_element_type=jnp.float32) + br_ref[...]
    m = jnp.max(logits, axis=-1, keepdims=True)
    ex = jnp.exp(logits - m)
    probs = ex / jnp.sum(ex, axis=-1, keepdims=True)
    col = lax.broadcasted_iota(jnp.int32, probs.shape, 1)
    big = jnp.int32(E + 1)
    m1 = jnp.max(probs, axis=-1, keepdims=True)
    a1 = jnp.min(jnp.where(probs == m1, col, big), axis=-1, keepdims=True)
    p2 = jnp.where(col == a1, -jnp.inf, probs)
    m2 = jnp.max(p2, axis=-1, keepdims=True)
    a2 = jnp.min(jnp.where(p2 == m2, col, big), axis=-1, keepdims=True)
    denom = m1 + m2
    w0_ref[...] = m1 / denom
    w1_ref[...] = m2 / denom

    Mm = ((col == a1) | (col == a2)).astype(jnp.bfloat16)
    r0 = lax.broadcasted_iota(jnp.int32, (T, T), 0)
    r1 = lax.broadcasted_iota(jnp.int32, (T, T), 1)
    L = (r1 < r0).astype(jnp.bfloat16)
    cnt_excl = jnp.dot(L, Mm, preferred_element_type=jnp.float32)

    counts = jnp.sum(Mm.astype(jnp.float32), axis=0, keepdims=True)
    nb = jnp.floor((counts + (BLK - 1)) / BLK)
    ecol0 = lax.broadcasted_iota(jnp.int32, (E, E), 0)
    ecol1 = lax.broadcasted_iota(jnp.int32, (E, E), 1)
    SU = (ecol0 < ecol1).astype(jnp.float32)
    offs_row = BLK * jnp.dot(nb, SU,
                             preferred_element_type=jnp.float32)

    posf0 = jnp.sum(jnp.where(col == a1, cnt_excl + offs_row, 0.0),
                    axis=-1, keepdims=True)
    posf1 = jnp.sum(jnp.where(col == a2, cnt_excl + offs_row, 0.0),
                    axis=-1, keepdims=True)
    pos0_ref[...] = posf0.astype(jnp.int32)
    pos1_ref[...] = posf1.astype(jnp.int32)

    IdE = (ecol0 == ecol1).astype(jnp.float32)
    SL = (ecol1 < ecol0).astype(jnp.float32)
    nbc = lax.dot_general(IdE, nb, (((1,), (1,)), ((), ())),
                          preferred_element_type=jnp.float32)
    startblk = jnp.dot(SL, nbc, preferred_element_type=jnp.float32)
    total = jnp.sum(nb, axis=-1, keepdims=True)
    bio = lax.broadcasted_iota(jnp.int32, (1, MAXB), 1).astype(jnp.float32)
    bclamp = jnp.minimum(bio, total - 1.0)
    owners = jnp.sum((startblk <= bclamp).astype(jnp.float32),
                     axis=0, keepdims=True)
    be_ref[...] = (owners - 1.0).astype(jnp.int32)
    nb_ref[...] = total.astype(jnp.int32)


def _router(x2d, Wr, br2):
    T, D = x2d.shape
    outs = pl.pallas_call(
        _router_kernel,
        grid=(1,),
        in_specs=[
            pl.BlockSpec((T, D), lambda i: (0, 0)),
            pl.BlockSpec((D, E), lambda i: (0, 0)),
            pl.BlockSpec((1, E), lambda i: (0, 0)),
        ],
        out_specs=[
            pl.BlockSpec((T, 1), lambda i: (0, 0)),
            pl.BlockSpec((T, 1), lambda i: (0, 0)),
            pl.BlockSpec((T, 1), lambda i: (0, 0)),
            pl.BlockSpec((T, 1), lambda i: (0, 0)),
            pl.BlockSpec((1, MAXB), lambda i: (0, 0)),
            pl.BlockSpec((1, 1), lambda i: (0, 0)),
        ],
        out_shape=[
            jax.ShapeDtypeStruct((T, 1), jnp.int32),
            jax.ShapeDtypeStruct((T, 1), jnp.int32),
            jax.ShapeDtypeStruct((T, 1), jnp.float32),
            jax.ShapeDtypeStruct((T, 1), jnp.float32),
            jax.ShapeDtypeStruct((1, MAXB), jnp.int32),
            jax.ShapeDtypeStruct((1, 1), jnp.int32),
        ],
    )(x2d, Wr, br2)
    return outs


def _sc_scatter(x2d, i0, i1, pbuf):
    T, D = x2d.shape
    bpw = T // NW
    mesh = plsc.VectorSubcoreMesh(core_axis_name="c", subcore_axis_name="s")

    @functools.partial(
        pl.kernel, mesh=mesh,
        out_type=jax.ShapeDtypeStruct((pbuf, D), jnp.float32),
        scratch_types=[
            pltpu.VMEM((bpw,), jnp.int32),
            pltpu.VMEM((bpw,), jnp.int32),
            pltpu.VMEM((bpw, D), jnp.float32),
            pltpu.SemaphoreType.DMA,
        ],
    )
    def scat(x_hbm, i0_hbm, i1_hbm, xg_hbm, i0_v, i1_v, rows_v, sem):
        wid = lax.axis_index("s") * 2 + lax.axis_index("c")
        base = wid * bpw
        pltpu.sync_copy(i0_hbm.at[wid], i0_v)
        pltpu.sync_copy(i1_hbm.at[wid], i1_v)
        pltpu.sync_copy(x_hbm.at[pl.ds(base, bpw)], rows_v)
        pltpu.async_copy(rows_v, xg_hbm.at[i0_v], sem).wait()
        pltpu.async_copy(rows_v, xg_hbm.at[i1_v], sem).wait()

    return scat(x2d, i0, i1)


def _sc_gather(yg, i0, i1):
    pbuf, D = yg.shape
    T = i0.shape[0] * i0.shape[1]
    bpw = T // NW
    mesh = plsc.VectorSubcoreMesh(core_axis_name="c", subcore_axis_name="s")

    @functools.partial(
        pl.kernel, mesh=mesh,
        out_type=(jax.ShapeDtypeStruct((T, D), jnp.float32),
                  jax.ShapeDtypeStruct((T, D), jnp.float32)),
        scratch_types=[
            pltpu.VMEM((bpw,), jnp.int32),
            pltpu.VMEM((bpw,), jnp.int32),
            pltpu.VMEM((bpw, D), jnp.float32),
            pltpu.SemaphoreType.DMA,
        ],
    )
    def gath(yg_hbm, i0_hbm, i1_hbm, o0_hbm, o1_hbm, i0_v, i1_v, rows_v, sem):
        wid = lax.axis_index("s") * 2 + lax.axis_index("c")
        base = wid * bpw
        pltpu.sync_copy(i0_hbm.at[wid], i0_v)
        pltpu.sync_copy(i1_hbm.at[wid], i1_v)
        pltpu.async_copy(yg_hbm.at[i0_v], rows_v, sem).wait()
        pltpu.sync_copy(rows_v, o0_hbm.at[pl.ds(base, bpw)])
        pltpu.async_copy(yg_hbm.at[i1_v], rows_v, sem).wait()
        pltpu.sync_copy(rows_v, o1_hbm.at[pl.ds(base, bpw)])

    return gath(yg, i0, i1)


def _ffn_kernel(be_ref, nb_ref, xg_ref, w1_ref, b1_ref, w2_ref, b2_ref,
                yg_ref):
    b = pl.program_id(0)
    j = pl.program_id(1)

    @pl.when(b < nb_ref[0])
    def _():
        xb = xg_ref[...].astype(jnp.bfloat16)
        h = jnp.dot(xb, w1_ref[0].astype(jnp.bfloat16),
                    preferred_element_type=jnp.float32) + b1_ref[0]
        hb = jnp.maximum(h, 0.0).astype(jnp.bfloat16)
        y = jnp.dot(hb, w2_ref[0].astype(jnp.bfloat16),
                    preferred_element_type=jnp.float32)

        @pl.when(j == 0)
        def _init():
            yg_ref[...] = y + b2_ref[0]

        @pl.when(j != 0)
        def _acc():
            yg_ref[...] = yg_ref[...] + y


def _ffn(xg, W1, b1r, W2, b2r, be, nb):
    pbuf, D = xg.shape
    F = W1.shape[2]
    FC = F // NF2

    def _wj(b, j, nb):
        return jnp.where(b < nb[0], j, NF2 - 1)

    def _bc(b, nb):
        return jnp.minimum(b, nb[0] - 1)

    yg = pl.pallas_call(
        _ffn_kernel,
        grid_spec=pltpu.PrefetchScalarGridSpec(
            num_scalar_prefetch=2,
            grid=(MAXB, NF2),
            in_specs=[
                pl.BlockSpec((BLK, D), lambda b, j, be, nb: (_bc(b, nb), 0)),
                pl.BlockSpec((1, D, FC),
                             lambda b, j, be, nb: (be[b], 0, _wj(b, j, nb))),
                pl.BlockSpec((1, 1, FC),
                             lambda b, j, be, nb: (be[b], 0, _wj(b, j, nb))),
                pl.BlockSpec((1, FC, D),
                             lambda b, j, be, nb: (be[b], _wj(b, j, nb), 0)),
                pl.BlockSpec((1, 1, D), lambda b, j, be, nb: (be[b], 0, 0)),
            ],
            out_specs=pl.BlockSpec((BLK, D),
                                   lambda b, j, be, nb: (_bc(b, nb), 0)),
        ),
        out_shape=jax.ShapeDtypeStruct((pbuf, D), jnp.float32),
        compiler_params=pltpu.CompilerParams(
            dimension_semantics=("arbitrary", "arbitrary"),
        ),
    )(be, nb, xg, W1, b1r, W2, b2r)
    return yg


def _combine_kernel(y0_ref, y1_ref, w0_ref, w1_ref, out_ref):
    out_ref[...] = y0_ref[...] * w0_ref[...] + y1_ref[...] * w1_ref[...]


def _combine(yg0, yg1, w0, w1):
    T, D = yg0.shape
    BT = 512
    return pl.pallas_call(
        _combine_kernel,
        grid=(T // BT,),
        in_specs=[
            pl.BlockSpec((BT, D), lambda i: (i, 0)),
            pl.BlockSpec((BT, D), lambda i: (i, 0)),
            pl.BlockSpec((BT, 1), lambda i: (i, 0)),
            pl.BlockSpec((BT, 1), lambda i: (i, 0)),
        ],
        out_specs=pl.BlockSpec((BT, D), lambda i: (i, 0)),
        out_shape=jax.ShapeDtypeStruct((T, D), jnp.float32),
    )(yg0, yg1, w0, w1)


@jax.jit
def _moe(x2d, Wr, br2, W1, b1r, W2, b2r):
    T, D = x2d.shape
    pbuf = MAXB * BLK
    pos0, pos1, w0, w1, be, nb = _router(x2d, Wr, br2)
    i0 = pos0.reshape(NW, T // NW)
    i1 = pos1.reshape(NW, T // NW)
    xg = _sc_scatter(x2d, i0, i1, pbuf)
    yg = _ffn(xg, W1, b1r, W2, b2r, be.reshape(MAXB), nb.reshape(1))
    yg0, yg1 = _sc_gather(yg, i0, i1)
    return _combine(yg0, yg1, w0, w1)


def kernel(x, Wr, br, W1, b1, W2, b2):
    B, S, D = x.shape
    x2d = x.reshape(B * S, D)
    out = _moe(x2d, Wr, br.reshape(1, E),
               W1, b1.reshape(E, 1, -1), W2, b2.reshape(E, 1, -1))
    return out.reshape(B, S, D)
